# Initial kernel scaffold; baseline (speedup 1.0000x reference)
import jax
import jax.numpy as jnp
from jax.experimental import pallas as pl


def kernel(x, weight):
    raise NotImplementedError("write your pallas kernel here")


# speedup vs baseline: 1.1853x; 1.1853x over previous
import functools

import jax
import jax.numpy as jnp
from jax import lax
from jax.experimental import pallas as pl
from jax.experimental.pallas import tpu as pltpu
from jax.experimental.pallas import tpu_sc as plsc

_SCALE = 10.0
_NC = 2
_NS = 16
_NW = _NC * _NS
_CH = 128


@functools.lru_cache(maxsize=None)
def _make_gather_scale(V, D, B):
  assert B % (_NW * _CH) == 0
  b_per_w = B // _NW
  n_ch = b_per_w // _CH
  mesh = plsc.VectorSubcoreMesh(core_axis_name="c", subcore_axis_name="s")

  @functools.partial(
      pl.kernel,
      mesh=mesh,
      out_type=jax.ShapeDtypeStruct((B, D), jnp.float32),
      scratch_types=[
          pltpu.VMEM((n_ch, _CH), jnp.int32),
          pltpu.VMEM((_CH, D), jnp.float32),
          pltpu.SemaphoreType.DMA,
      ],
      compiler_params=pltpu.CompilerParams(use_tc_tiling_on_sc=False),
  )
  def gather_scale(table_hbm, idx_hbm, out_hbm, idx_v, rows_v, sem):
    wid = lax.axis_index("s") * _NC + lax.axis_index("c")
    ch_base = wid * n_ch
    pltpu.sync_copy(idx_hbm.at[pl.ds(ch_base, n_ch)], idx_v)

    def chunk_body(c, carry):
      pltpu.async_copy(table_hbm.at[idx_v.at[c]], rows_v, sem).wait()

      def row_body(j, carry2):
        for h in range(D // 16):
          rows_v[j, pl.ds(h * 16, 16)] = rows_v[j, pl.ds(h * 16, 16)] * _SCALE
        return carry2

      lax.fori_loop(0, _CH, row_body, 0, unroll=2)

      pltpu.sync_copy(rows_v, out_hbm.at[pl.ds((ch_base + c) * _CH, _CH)])
      return carry

    lax.fori_loop(0, n_ch, chunk_body, 0)

  return gather_scale


def kernel(x, weight):
  S0, S1 = x.shape
  V, D = weight.shape
  B = S0 * S1
  idx = x.reshape(B // _CH, _CH).astype(jnp.int32)
  out = _make_gather_scale(V, D, B)(weight, idx)
  return out.reshape(S0, S1, D)


# speedup vs baseline: 1.2711x; 1.0725x over previous
import functools

import jax
import jax.numpy as jnp
from jax import lax
from jax.experimental import pallas as pl
from jax.experimental.pallas import tpu as pltpu
from jax.experimental.pallas import tpu_sc as plsc

_SCALE = 10.0
_NC = 2
_NS = 16
_NW = _NC * _NS
_CH = 128
_NBUF = 8


@functools.lru_cache(maxsize=None)
def _make_gather_scale(V, D, B):
  assert B % (_NW * _CH) == 0
  b_per_w = B // _NW
  n_ch = b_per_w // _CH
  assert n_ch % _NBUF == 0 and n_ch >= 2 * _NBUF
  mesh = plsc.VectorSubcoreMesh(core_axis_name="c", subcore_axis_name="s")

  @functools.partial(
      pl.kernel,
      mesh=mesh,
      out_type=jax.ShapeDtypeStruct((B, D), jnp.float32),
      scratch_types=[
          pltpu.VMEM((n_ch, _CH), jnp.int32),
          pltpu.VMEM((_NBUF, _CH, D), jnp.float32),
          pltpu.VMEM((_NBUF, _CH, D), jnp.float32),
          [pltpu.SemaphoreType.DMA] * _NBUF,
          [pltpu.SemaphoreType.DMA] * _NBUF,
      ],
      compiler_params=pltpu.CompilerParams(use_tc_tiling_on_sc=False),
  )
  def gather_scale(table_hbm, idx_hbm, out_hbm, idx_v, in_b, out_b,
                   in_sems, out_sems):
    wid = lax.axis_index("s") * _NC + lax.axis_index("c")
    ch_base = wid * n_ch
    pltpu.sync_copy(idx_hbm.at[pl.ds(ch_base, n_ch)], idx_v)

    def gather(c, bi):
      return pltpu.async_copy(
          table_hbm.at[idx_v.at[c]], in_b.at[bi], in_sems[bi])

    def writeback(c, bi):
      return pltpu.async_copy(
          out_b.at[bi], out_hbm.at[pl.ds((ch_base + c) * _CH, _CH)],
          out_sems[bi])

    for bi in range(_NBUF):
      gather(bi, bi)

    def outer(c0, carry):
      for bi in range(_NBUF):
        c = c0 + bi
        pltpu.make_async_copy(
            table_hbm.at[idx_v.at[c]], in_b.at[bi], in_sems[bi]).wait()
        @pl.when(c >= _NBUF)
        def _():
          pltpu.make_async_copy(
              out_b.at[bi], out_hbm.at[pl.ds((ch_base + c) * _CH, _CH)],
              out_sems[bi]).wait()

        def row_body(j, carry2):
          for h in range(D // 16):
            out_b[bi, j, pl.ds(h * 16, 16)] = (
                in_b[bi, j, pl.ds(h * 16, 16)] * _SCALE)
          return carry2

        lax.fori_loop(0, _CH, row_body, 0, unroll=4)

        writeback(c, bi)

        @pl.when(c + _NBUF < n_ch)
        def _():
          gather(c + _NBUF, bi)
      return carry

    lax.fori_loop(0, n_ch // _NBUF, lambda i, cr: outer(i * _NBUF, cr), 0)

    for bi in range(_NBUF):
      c = n_ch - _NBUF + bi
      pltpu.make_async_copy(
          out_b.at[bi], out_hbm.at[pl.ds((ch_base + c) * _CH, _CH)],
          out_sems[bi]).wait()

  return gather_scale


def kernel(x, weight):
  S0, S1 = x.shape
  V, D = weight.shape
  B = S0 * S1
  idx = x.reshape(B // _CH, _CH).astype(jnp.int32)
  out = _make_gather_scale(V, D, B)(weight, idx)
  return out.reshape(S0, S1, D)
